# baseline (speedup 1.0000x reference)
import functools

import numpy as np

import jax
import jax.numpy as jnp
from jax.experimental import pallas as pl
from jax.experimental.pallas import tpu as pltpu
from jax.sharding import Mesh, PartitionSpec as P
from jax import shard_map

NUM_EMBEDDINGS = 1024
EMBEDDING_DIM = 30
COMMITMENT_COST = 0.25

BLOCK_ROWS = 1152


def _vq_body(x_ref, ea_ref,
             dist_ref, enc_ref, quant_ref, idx_ref, sse_ref, counts_ref):
    x = x_ref[...]
    ea = ea_ref[...]
    b, d = x.shape
    k = ea.shape[0]

    emb = ea[:, :d]
    esq = ea[:, d]
    xsq = jnp.sum(x * x, axis=1, keepdims=True)
    dot = jax.lax.dot_general(x, emb, (((1,), (1,)), ((), ())),
                              preferred_element_type=jnp.float32)
    dist = xsq + esq[None, :] - 2.0 * dot
    dist_ref[...] = dist

    min_val = jnp.min(dist, axis=1, keepdims=True)
    iota = jax.lax.broadcasted_iota(jnp.int32, (b, k), 1)
    idx = jnp.min(jnp.where(dist == min_val, iota, k), axis=1)
    idx_ref[...] = idx.reshape(1, b // 128, 128).astype(jnp.int32)

    onehot = (iota == idx[:, None]).astype(jnp.float32)
    enc_ref[...] = onehot

    quant = jax.lax.dot_general(onehot, emb, (((1,), (0,)), ((), ())),
                                preferred_element_type=jnp.float32)
    quant_ref[...] = x + (quant - x)
    diff = quant - x

    counts = jax.lax.dot_general(jnp.ones((1, b), jnp.float32), onehot,
                                 (((1,), (0,)), ((), ())),
                                 preferred_element_type=jnp.float32)
    sse_ref[...] = jnp.sum(diff * diff).reshape(1, 1, 1)
    counts_ref[...] = counts.reshape(1, 1, k)


def _vq_shard(n_total, sharded, x_loc, embedding):
    flat = x_loc.reshape(-1, EMBEDDING_DIM)
    n = flat.shape[0]
    k = NUM_EMBEDDINGS
    block = min(BLOCK_ROWS, n)
    num_blocks = n // block

    esq = jnp.sum(embedding * embedding, axis=1, keepdims=True)
    ea = jnp.concatenate(
        [embedding, esq, jnp.ones((k, 1), jnp.float32)], axis=1)

    dist, enc, quant, idx, sse, counts = pl.pallas_call(
        _vq_body,
        grid=(num_blocks,),
        in_specs=[
            pl.BlockSpec((block, EMBEDDING_DIM), lambda i: (i, 0)),
            pl.BlockSpec((k, EMBEDDING_DIM + 2), lambda i: (0, 0)),
        ],
        out_specs=[
            pl.BlockSpec((block, k), lambda i: (i, 0)),
            pl.BlockSpec((block, k), lambda i: (i, 0)),
            pl.BlockSpec((block, EMBEDDING_DIM), lambda i: (i, 0)),
            pl.BlockSpec((1, block // 128, 128), lambda i: (i, 0, 0)),
            pl.BlockSpec((1, 1, 1), lambda i: (i, 0, 0)),
            pl.BlockSpec((1, 1, k), lambda i: (i, 0, 0)),
        ],
        out_shape=[
            jax.ShapeDtypeStruct((n, k), jnp.float32),
            jax.ShapeDtypeStruct((n, k), jnp.float32),
            jax.ShapeDtypeStruct((n, EMBEDDING_DIM), jnp.float32),
            jax.ShapeDtypeStruct((num_blocks, block // 128, 128), jnp.int32),
            jax.ShapeDtypeStruct((num_blocks, 1, 1), jnp.float32),
            jax.ShapeDtypeStruct((num_blocks, 1, k), jnp.float32),
        ],
        compiler_params=pltpu.CompilerParams(
            dimension_semantics=("parallel",),
            vmem_limit_bytes=63 * 1024 * 1024,
        ),
    )(flat, ea)
    sse = jnp.sum(sse, axis=0)
    counts = jnp.sum(counts, axis=0)

    if sharded:
        sse = jax.lax.psum(sse, "x")
        counts = jax.lax.psum(counts, "x")
    loss = sse[0, 0] / (n_total * EMBEDDING_DIM)
    avg = counts[0] / n_total
    perp = jnp.exp(-jnp.sum(avg * jnp.log(avg + 1e-10)))
    return dist, enc, quant, idx, loss, perp


def kernel(inputs, embedding):
    x = jnp.transpose(inputs, (2, 1, 0))
    time, c, batch = x.shape
    n_total = time * c

    devs = jax.devices()
    n_dev = 2 if len(devs) >= 2 and time % 2 == 0 else 1

    if n_dev == 1:
        dist, enc, quant, idx, loss, perp = _vq_shard(
            n_total, False, x, embedding)
    else:
        mesh = Mesh(np.array(devs[:n_dev]), ("x",))
        shard_fn = shard_map(
            functools.partial(_vq_shard, n_total, True),
            mesh=mesh,
            in_specs=(P("x", None, None), P(None, None)),
            out_specs=(P("x", None), P("x", None), P("x", None),
                       P("x", None), P(), P()),
            check_vma=False,
        )
        dist, enc, quant, idx, loss, perp = shard_fn(x, embedding)

    quantized_st = jnp.transpose(quant.reshape(time, c, batch), (2, 1, 0))
    return (loss,
            quantized_st,
            perp,
            enc.reshape(batch, c, -1),
            dist.reshape(batch, c, -1),
            idx.reshape(-1, 1))


# speedup vs baseline: 1.1555x; 1.1555x over previous
import functools

import numpy as np

import jax
import jax.numpy as jnp
from jax.experimental import pallas as pl
from jax.experimental.pallas import tpu as pltpu
from jax.sharding import Mesh, PartitionSpec as P
from jax import shard_map

NUM_EMBEDDINGS = 1024
EMBEDDING_DIM = 30
COMMITMENT_COST = 0.25

BLOCK_ROWS = 2560


def _vq_body(x_ref, ea_ref,
             dist_ref, enc_ref, quant_ref, idx_ref, sse_ref, counts_ref):
    x = x_ref[...]
    ea = ea_ref[...]
    b, d = x.shape
    k = ea.shape[0]

    emb = ea[:, :d]
    esq = ea[:, d]
    xsq = jnp.sum(x * x, axis=1, keepdims=True)
    dot = jax.lax.dot_general(x, emb, (((1,), (1,)), ((), ())),
                              preferred_element_type=jnp.float32)
    dist = xsq + esq[None, :] - 2.0 * dot
    dist_ref[...] = dist

    min_val = jnp.min(dist, axis=1, keepdims=True)
    iota = jax.lax.broadcasted_iota(jnp.int32, (b, k), 1)
    idx = jnp.min(jnp.where(dist == min_val, iota, k), axis=1)
    idx_ref[...] = idx.reshape(1, b // 128, 128).astype(jnp.int32)

    onehot = (iota == idx[:, None]).astype(jnp.float32)
    enc_ref[...] = onehot

    quant = jax.lax.dot_general(onehot, emb, (((1,), (0,)), ((), ())),
                                preferred_element_type=jnp.float32)
    quant_ref[...] = x + (quant - x)
    diff = quant - x

    counts = jax.lax.dot_general(jnp.ones((1, b), jnp.float32), onehot,
                                 (((1,), (0,)), ((), ())),
                                 preferred_element_type=jnp.float32)
    sse_ref[...] = jnp.sum(diff * diff).reshape(1, 1, 1)
    counts_ref[...] = counts.reshape(1, 1, k)


def _vq_shard(n_total, sharded, x_loc, embedding):
    flat = x_loc.reshape(-1, EMBEDDING_DIM)
    n = flat.shape[0]
    k = NUM_EMBEDDINGS
    block = min(BLOCK_ROWS, n)
    num_blocks = n // block

    esq = jnp.sum(embedding * embedding, axis=1, keepdims=True)
    ea = jnp.concatenate(
        [embedding, esq, jnp.ones((k, 1), jnp.float32)], axis=1)

    dist, enc, quant, idx, sse, counts = pl.pallas_call(
        _vq_body,
        grid=(num_blocks,),
        in_specs=[
            pl.BlockSpec((block, EMBEDDING_DIM), lambda i: (i, 0)),
            pl.BlockSpec((k, EMBEDDING_DIM + 2), lambda i: (0, 0)),
        ],
        out_specs=[
            pl.BlockSpec((block, k), lambda i: (i, 0)),
            pl.BlockSpec((block, k), lambda i: (i, 0)),
            pl.BlockSpec((block, EMBEDDING_DIM), lambda i: (i, 0)),
            pl.BlockSpec((1, block // 128, 128), lambda i: (i, 0, 0)),
            pl.BlockSpec((1, 1, 1), lambda i: (i, 0, 0)),
            pl.BlockSpec((1, 1, k), lambda i: (i, 0, 0)),
        ],
        out_shape=[
            jax.ShapeDtypeStruct((n, k), jnp.float32),
            jax.ShapeDtypeStruct((n, k), jnp.float32),
            jax.ShapeDtypeStruct((n, EMBEDDING_DIM), jnp.float32),
            jax.ShapeDtypeStruct((num_blocks, block // 128, 128), jnp.int32),
            jax.ShapeDtypeStruct((num_blocks, 1, 1), jnp.float32),
            jax.ShapeDtypeStruct((num_blocks, 1, k), jnp.float32),
        ],
        compiler_params=pltpu.CompilerParams(
            dimension_semantics=("parallel",),
            vmem_limit_bytes=63 * 1024 * 1024,
        ),
    )(flat, ea)
    sse = jnp.sum(sse, axis=0)
    counts = jnp.sum(counts, axis=0)

    if sharded:
        sse = jax.lax.psum(sse, "x")
        counts = jax.lax.psum(counts, "x")
    loss = sse[0, 0] / (n_total * EMBEDDING_DIM)
    avg = counts[0] / n_total
    perp = jnp.exp(-jnp.sum(avg * jnp.log(avg + 1e-10)))
    return dist, enc, quant, idx, loss, perp


def kernel(inputs, embedding):
    x = jnp.transpose(inputs, (2, 1, 0))
    time, c, batch = x.shape
    n_total = time * c

    devs = jax.devices()
    n_dev = 2 if len(devs) >= 2 and time % 2 == 0 else 1

    if n_dev == 1:
        dist, enc, quant, idx, loss, perp = _vq_shard(
            n_total, False, x, embedding)
    else:
        mesh = Mesh(np.array(devs[:n_dev]), ("x",))
        shard_fn = shard_map(
            functools.partial(_vq_shard, n_total, True),
            mesh=mesh,
            in_specs=(P("x", None, None), P(None, None)),
            out_specs=(P("x", None), P("x", None), P("x", None),
                       P("x", None), P(), P()),
            check_vma=False,
        )
        dist, enc, quant, idx, loss, perp = shard_fn(x, embedding)

    quantized_st = jnp.transpose(quant.reshape(time, c, batch), (2, 1, 0))
    return (loss,
            quantized_st,
            perp,
            enc.reshape(batch, c, -1),
            dist.reshape(batch, c, -1),
            idx.reshape(-1, 1))
